# baseline (speedup 1.0000x reference)
import functools

import jax
import jax.numpy as jnp
from jax import lax
from jax.experimental import pallas as pl
from jax.experimental.pallas import tpu as pltpu
from jax.experimental.pallas import tpu_sc as plsc

_B, _H, _W, _C = 8, 224, 224, 96
_N = _B * _H * _W
_NW = 32
_PT = _N // _NW
_PIX = 32
_NB = _PT // _PIX
_G = _PIX // 16
_R = 4 * _PIX
_CU = 8
_CH = 64


@functools.partial(
    pl.kernel,
    out_type=jax.ShapeDtypeStruct((_N, _C), jnp.float32),
    mesh=plsc.VectorSubcoreMesh(core_axis_name="c", subcore_axis_name="s"),
    compiler_params=pltpu.CompilerParams(
        needs_layout_passes=False, use_tc_tiling_on_sc=False),
    scratch_types=[
        pltpu.VMEM((_PT,), jnp.float32),
        pltpu.VMEM((_PT,), jnp.float32),
        pltpu.VMEM((2, _R + 2), jnp.int32),
        pltpu.VMEM((2, _R), jnp.int32),
        pltpu.VMEM((2, _R), jnp.int32),
        pltpu.VMEM((2, _R), jnp.float32),
        pltpu.VMEM((2 * _R, _C), jnp.float32),
        pltpu.VMEM((2 * _PIX, _C), jnp.float32),
        pltpu.SMEM((2,), jnp.int32),
        pltpu.SemaphoreType.DMA,
        pltpu.SemaphoreType.DMA,
        pltpu.SemaphoreType.DMA,
        pltpu.SemaphoreType.DMA,
    ],
)
def _sc_bilinear(img_hbm, xs_hbm, ys_hbm, out_hbm,
                 xs_v, ys_v, seq_v, pos_v, idxc_v, w_v, buf_v, out_v, nch_s,
                 gsem0, gsem1, osem0, osem1):
    gsem = (gsem0, gsem1)
    osem = (osem0, osem1)
    wid = lax.axis_index("c") * 16 + lax.axis_index("s")
    tile_base = wid * _PT
    img_base = (wid // 4) * (_H * _W)

    pltpu.sync_copy(xs_hbm.at[pl.ds(tile_base, _PT)], xs_v)
    pltpu.sync_copy(ys_hbm.at[pl.ds(tile_base, _PT)], ys_v)

    iota = lax.iota(jnp.int32, 16)

    def prep(k, s):
        srow = jnp.full((16,), s, jnp.int32)
        seq_v[s, pl.ds(0, 16)] = jnp.full((16,), -1, jnp.int32)
        for g in range(_G):
            off = g * 64
            xv = xs_v[pl.ds(k * _PIX + g * 16, 16)]
            yv = ys_v[pl.ds(k * _PIX + g * 16, 16)]
            x = (0.5 * (xv + 1.0)) * jnp.float32(_W)
            y = (0.5 * (yv + 1.0)) * jnp.float32(_H)
            x0r = x.astype(jnp.int32)
            y0r = y.astype(jnp.int32)
            x0 = jnp.clip(x0r, 0, _W - 1)
            x1 = jnp.clip(x0r + 1, 0, _W - 1)
            y0 = jnp.clip(y0r, 0, _H - 1)
            y1 = jnp.clip(y0r + 1, 0, _H - 1)
            x0f = x0.astype(jnp.float32)
            x1f = x1.astype(jnp.float32)
            y0f = y0.astype(jnp.float32)
            y1f = y1.astype(jnp.float32)
            ra = img_base + y0 * _W + x0
            rb = img_base + y1 * _W + x0
            dx = x1 - x0
            col_a = 1 + 2 * (g * 16) + 2 * iota
            plsc.store_scatter(seq_v, [srow, col_a], ra)
            plsc.store_scatter(seq_v, [srow, col_a + 1], ra + dx)
            plsc.store_scatter(seq_v, [srow, 64 + col_a], rb)
            plsc.store_scatter(seq_v, [srow, 64 + col_a + 1], rb + dx)
            w_v[s, pl.ds(off, 16)] = (x1f - x) * (y1f - y)
            w_v[s, pl.ds(off + 16, 16)] = (x1f - x) * (y - y0f)
            w_v[s, pl.ds(off + 32, 16)] = (x - x0f) * (y1f - y)
            w_v[s, pl.ds(off + 48, 16)] = (x - x0f) * (y - y0f)
        zpad = jnp.zeros((16,), jnp.int32)
        for t in range(_R // 16):
            idxc_v[s, pl.ds(16 * t, 16)] = zpad
        carry = jnp.int32(0)
        for j in range(_R // 16):
            sv = seq_v[s, pl.ds(1 + j * 16, 16)]
            pv = seq_v[s, pl.ds(j * 16, 16)]
            fb = sv != pv
            fi = jnp.where(fb, jnp.int32(1), jnp.int32(0))
            cum = plsc.cumsum(fi) + carry
            pos_v[s, pl.ds(j * 16, 16)] = cum - 1
            plsc.store_compressed(idxc_v.at[s, pl.ds(carry, 16)], sv, mask=fb)
            carry = carry + jnp.sum(fi)
        n = carry
        nch_s[s] = n
        pltpu.async_copy(
            img_hbm.at[idxc_v.at[s, pl.ds(0, _CH)]],
            buf_v.at[pl.ds(s * _R, _CH)], gsem[s])

        @pl.when(n > _CH)
        def _():
            pltpu.async_copy(
                img_hbm.at[idxc_v.at[s, pl.ds(_CH, _CH)]],
                buf_v.at[pl.ds(s * _R + _CH, _CH)], gsem[s])

    def wait_gathers(s):
        pltpu.make_async_copy(
            img_hbm.at[idxc_v.at[s, pl.ds(0, _CH)]],
            buf_v.at[pl.ds(s * _R, _CH)], gsem[s]).wait()

        @pl.when(nch_s[s] > _CH)
        def _():
            pltpu.make_async_copy(
                img_hbm.at[idxc_v.at[s, pl.ds(0, _CH)]],
                buf_v.at[pl.ds(s * _R, _CH)], gsem[s]).wait()

    def combine(s):
        srow = jnp.full((16,), s, jnp.int32)
        for g in range(_G):
            off = g * 64
            wa = w_v[s, pl.ds(off, 16)]
            wb = w_v[s, pl.ds(off + 16, 16)]
            wc = w_v[s, pl.ds(off + 32, 16)]
            wd = w_v[s, pl.ds(off + 48, 16)]
            e_a = 2 * (g * 16) + 2 * iota
            ra = s * _R + plsc.load_gather(pos_v, [srow, e_a])
            rc = s * _R + plsc.load_gather(pos_v, [srow, e_a + 1])
            rb = s * _R + plsc.load_gather(pos_v, [srow, 64 + e_a])
            rd = s * _R + plsc.load_gather(pos_v, [srow, 64 + e_a + 1])
            po = s * _PIX + g * 16 + iota

            def chan(ci, acc, wa=wa, wb=wb, wc=wc, wd=wd,
                     ra=ra, rb=rb, rc=rc, rd=rd, po=po):
                for cc in range(_CU):
                    c = ci * _CU + cc
                    cs = jnp.full((16,), c, jnp.int32)
                    pa = plsc.load_gather(buf_v, [ra, cs])
                    pb = plsc.load_gather(buf_v, [rb, cs])
                    pc = plsc.load_gather(buf_v, [rc, cs])
                    pd = plsc.load_gather(buf_v, [rd, cs])
                    o = wa * pa + wb * pb + wc * pc + wd * pd
                    plsc.store_scatter(out_v, [po, cs], o)
                return acc

            lax.fori_loop(0, _C // _CU, chan, 0)

    prep(0, 0)

    def body2(k2, carry2):
        for s in range(2):
            k = 2 * k2 + s

            @pl.when(k + 1 < _NB)
            def _():
                prep(k + 1, s ^ 1)

            wait_gathers(s)

            @pl.when(k >= 2)
            def _():
                pltpu.make_async_copy(
                    out_v.at[pl.ds(s * _PIX, _PIX)],
                    out_hbm.at[pl.ds(tile_base, _PIX)], osem[s]).wait()

            combine(s)
            pltpu.async_copy(
                out_v.at[pl.ds(s * _PIX, _PIX)],
                out_hbm.at[pl.ds(tile_base + k * _PIX, _PIX)], osem[s])
        return carry2

    lax.fori_loop(0, _NB // 2, body2, 0)

    for s in range(2):
        pltpu.make_async_copy(
            out_v.at[pl.ds(s * _PIX, _PIX)],
            out_hbm.at[pl.ds(tile_base, _PIX)], osem[s]).wait()


def kernel(X, transformation):
    Bx, H, W, C = X.shape
    Hout, Wout = 224, 224
    flat_out = Hout * Wout
    x_lin = jnp.linspace(-1.0, 1.0, Wout)
    y_lin = jnp.linspace(-1.0, 1.0, Hout)
    xg, yg = jnp.meshgrid(x_lin, y_lin)
    grid = jnp.concatenate([xg.ravel(), yg.ravel(), jnp.ones(flat_out)], axis=0)
    grid = grid.reshape(3, flat_out).astype(jnp.float32)
    T = transformation.reshape(Bx, 2, 3)
    sampled = jnp.einsum('bij,jk->bik', T, grid)
    xs = sampled[:, 0, :].reshape(-1)
    ys = sampled[:, 1, :].reshape(-1)
    img = X.reshape(-1, C)
    out = _sc_bilinear(img, xs, ys)
    return out.reshape(Bx, Hout, Wout, C)


# speedup vs baseline: 2.0341x; 2.0341x over previous
import functools

import jax
import jax.numpy as jnp
from jax import lax
from jax.experimental import pallas as pl
from jax.experimental.pallas import tpu as pltpu
from jax.experimental.pallas import tpu_sc as plsc

_B, _H, _W, _C = 8, 224, 224, 96
_N = _B * _H * _W
_NW = 32
_PT = _N // _NW
_PIX = 32
_NB = _PT // _PIX
_G = _PIX // 16
_R = 4 * _PIX
_CU = 8


@functools.partial(
    pl.kernel,
    out_type=jax.ShapeDtypeStruct((_N, _C), jnp.float32),
    mesh=plsc.VectorSubcoreMesh(core_axis_name="c", subcore_axis_name="s"),
    compiler_params=pltpu.CompilerParams(
        needs_layout_passes=False, use_tc_tiling_on_sc=False),
    scratch_types=[
        pltpu.VMEM((_PT,), jnp.float32),
        pltpu.VMEM((_PT,), jnp.float32),
        pltpu.VMEM((2, _R), jnp.int32),
        pltpu.VMEM((2, _R), jnp.float32),
        pltpu.VMEM((2 * _R, _C), jnp.float32),
        pltpu.VMEM((2 * _PIX, _C), jnp.float32),
        pltpu.SemaphoreType.DMA,
        pltpu.SemaphoreType.DMA,
        pltpu.SemaphoreType.DMA,
        pltpu.SemaphoreType.DMA,
    ],
)
def _sc_bilinear(img_hbm, xs_hbm, ys_hbm, out_hbm,
                 xs_v, ys_v, idx_v, w_v, buf_v, out_v,
                 gsem0, gsem1, osem0, osem1):
    gsem = (gsem0, gsem1)
    osem = (osem0, osem1)
    wid = lax.axis_index("c") * 16 + lax.axis_index("s")
    tile_base = wid * _PT
    img_base = (wid // 4) * (_H * _W)

    pltpu.sync_copy(xs_hbm.at[pl.ds(tile_base, _PT)], xs_v)
    pltpu.sync_copy(ys_hbm.at[pl.ds(tile_base, _PT)], ys_v)

    iota = lax.iota(jnp.int32, 16)

    def prep(k, s):
        for g in range(_G):
            off = g * 64
            xv = xs_v[pl.ds(k * _PIX + g * 16, 16)]
            yv = ys_v[pl.ds(k * _PIX + g * 16, 16)]
            x = (0.5 * (xv + 1.0)) * jnp.float32(_W)
            y = (0.5 * (yv + 1.0)) * jnp.float32(_H)
            x0r = x.astype(jnp.int32)
            y0r = y.astype(jnp.int32)
            x0 = jnp.clip(x0r, 0, _W - 1)
            x1 = jnp.clip(x0r + 1, 0, _W - 1)
            y0 = jnp.clip(y0r, 0, _H - 1)
            y1 = jnp.clip(y0r + 1, 0, _H - 1)
            x0f = x0.astype(jnp.float32)
            x1f = x1.astype(jnp.float32)
            y0f = y0.astype(jnp.float32)
            y1f = y1.astype(jnp.float32)
            ra = img_base + y0 * _W + x0
            rb = img_base + y1 * _W + x0
            dx = x1 - x0
            idx_v[s, pl.ds(off, 16)] = ra
            idx_v[s, pl.ds(off + 16, 16)] = rb
            idx_v[s, pl.ds(off + 32, 16)] = ra + dx
            idx_v[s, pl.ds(off + 48, 16)] = rb + dx
            w_v[s, pl.ds(off, 16)] = (x1f - x) * (y1f - y)
            w_v[s, pl.ds(off + 16, 16)] = (x1f - x) * (y - y0f)
            w_v[s, pl.ds(off + 32, 16)] = (x - x0f) * (y1f - y)
            w_v[s, pl.ds(off + 48, 16)] = (x - x0f) * (y - y0f)
        pltpu.async_copy(img_hbm.at[idx_v.at[s]],
                         buf_v.at[pl.ds(s * _R, _R)], gsem[s])

    def combine(s):
        srowv = jnp.full((16,), s, jnp.int32)

        def pix(p, acc):
            g64 = (p // 16) * 64 + (p % 16)
            g64v = jnp.full((16,), g64, jnp.int32)
            wa = plsc.load_gather(w_v, [srowv, g64v])
            wb = plsc.load_gather(w_v, [srowv, g64v + 16])
            wc = plsc.load_gather(w_v, [srowv, g64v + 32])
            wd = plsc.load_gather(w_v, [srowv, g64v + 48])
            row = s * _R + g64
            for cv in range(_C // 16):
                sl = pl.ds(cv * 16, 16)
                pa = buf_v[row, sl]
                pb = buf_v[row + 16, sl]
                pc = buf_v[row + 32, sl]
                pd = buf_v[row + 48, sl]
                out_v[s * _PIX + p, sl] = wa * pa + wb * pb + wc * pc + wd * pd
            return acc

        lax.fori_loop(0, _PIX, pix, 0)

    prep(0, 0)

    def body2(k2, carry):
        for s in range(2):
            k = 2 * k2 + s

            @pl.when(k + 1 < _NB)
            def _():
                prep(k + 1, s ^ 1)

            pltpu.make_async_copy(
                img_hbm.at[idx_v.at[s]],
                buf_v.at[pl.ds(s * _R, _R)], gsem[s]).wait()

            @pl.when(k >= 2)
            def _():
                pltpu.make_async_copy(
                    out_v.at[pl.ds(s * _PIX, _PIX)],
                    out_hbm.at[pl.ds(tile_base, _PIX)], osem[s]).wait()

            combine(s)
            pltpu.async_copy(
                out_v.at[pl.ds(s * _PIX, _PIX)],
                out_hbm.at[pl.ds(tile_base + k * _PIX, _PIX)], osem[s])
        return carry

    lax.fori_loop(0, _NB // 2, body2, 0)

    for s in range(2):
        pltpu.make_async_copy(
            out_v.at[pl.ds(s * _PIX, _PIX)],
            out_hbm.at[pl.ds(tile_base, _PIX)], osem[s]).wait()


def kernel(X, transformation):
    Bx, H, W, C = X.shape
    Hout, Wout = 224, 224
    flat_out = Hout * Wout
    x_lin = jnp.linspace(-1.0, 1.0, Wout)
    y_lin = jnp.linspace(-1.0, 1.0, Hout)
    xg, yg = jnp.meshgrid(x_lin, y_lin)
    grid = jnp.concatenate([xg.ravel(), yg.ravel(), jnp.ones(flat_out)], axis=0)
    grid = grid.reshape(3, flat_out).astype(jnp.float32)
    T = transformation.reshape(Bx, 2, 3)
    sampled = jnp.einsum('bij,jk->bik', T, grid)
    xs = sampled[:, 0, :].reshape(-1)
    ys = sampled[:, 1, :].reshape(-1)
    img = X.reshape(-1, C)
    out = _sc_bilinear(img, xs, ys)
    return out.reshape(Bx, Hout, Wout, C)
